# baseline (speedup 1.0000x reference)
import jax
import jax.numpy as jnp
from jax import lax
from jax.experimental import pallas as pl
from jax.experimental.pallas import tpu as pltpu

SEQ = 2048
BATCH = 16
D = 1024
BT = 128
NT = SEQ // BT


def _body(lens_ref, in_ref, out_ref):
    b = pl.program_id(0)
    t = pl.program_id(1)
    L = lens_ref[b]
    nb = (L + (BT - 1)) // BT

    @pl.when(t == 0)
    def _():
        out_ref[...] = jnp.zeros_like(out_ref)

    @pl.when(t < nb)
    def _():
        x = in_ref[...]
        rem = L - t * BT
        row = lax.broadcasted_iota(jnp.int32, (BT, D), 0)
        masked = jnp.where(row < rem, x, 0.0)
        out_ref[...] += jnp.sum(masked, axis=0)[None, None, :]

    @pl.when(t == nb - 1)
    def _():
        out_ref[...] = out_ref[...] / L.astype(jnp.float32)


def _seq_mean_tc(input2, lens):
    grid_spec = pltpu.PrefetchScalarGridSpec(
        num_scalar_prefetch=1,
        grid=(BATCH, NT),
        in_specs=[
            pl.BlockSpec(
                (BT, D),
                lambda b, t, lens: (
                    jnp.minimum(t, (lens[b] + (BT - 1)) // BT - 1), b),
            ),
        ],
        out_specs=pl.BlockSpec((1, 1, D), lambda b, t, lens: (b, 0, 0)),
    )
    return pl.pallas_call(
        _body,
        grid_spec=grid_spec,
        out_shape=jax.ShapeDtypeStruct((BATCH, 1, D), jnp.float32),
        compiler_params=pltpu.CompilerParams(
            dimension_semantics=("arbitrary", "arbitrary"),
        ),
    )(lens, input2)


def kernel(input, sequence_lengths):
    lens = sequence_lengths.astype(jnp.int32)
    input2 = input.reshape(SEQ, BATCH * D)
    return _seq_mean_tc(input2, lens).reshape(BATCH, D)


# speedup vs baseline: 1.0187x; 1.0187x over previous
import jax
import jax.numpy as jnp
from jax import lax
from jax.experimental import pallas as pl
from jax.experimental.pallas import tpu as pltpu

SEQ = 2048
BATCH = 16
D = 1024
BT = 128
NT = SEQ // BT


def _body(lens_ref, in_ref, out_ref, acc_ref):
    b = pl.program_id(0)
    t = pl.program_id(1)
    L = lens_ref[b]
    nb = (L + (BT - 1)) // BT
    rem = L - t * BT

    @pl.when(t == 0)
    def _():
        acc_ref[...] = jnp.zeros_like(acc_ref)

    @pl.when(rem >= BT)
    def _():
        x = in_ref[...].reshape(BT // 8, 8, D)
        acc_ref[...] += jnp.sum(x, axis=0)

    @pl.when((t < nb) & (rem < BT))
    def _():
        x = in_ref[...]
        row = lax.broadcasted_iota(jnp.int32, (BT, D), 0)
        masked = jnp.where(row < rem, x, 0.0).reshape(BT // 8, 8, D)
        acc_ref[...] += jnp.sum(masked, axis=0)

    @pl.when(t == nb - 1)
    def _():
        s = jnp.sum(acc_ref[...], axis=0)
        out_ref[...] = (s / L.astype(jnp.float32))[None, None, :]


def _seq_mean_tc(input2, lens):
    grid_spec = pltpu.PrefetchScalarGridSpec(
        num_scalar_prefetch=1,
        grid=(BATCH, NT),
        in_specs=[
            pl.BlockSpec(
                (BT, D),
                lambda b, t, lens: (
                    jnp.minimum(t, (lens[b] + (BT - 1)) // BT - 1), b),
            ),
        ],
        out_specs=pl.BlockSpec((1, 1, D), lambda b, t, lens: (b, 0, 0)),
        scratch_shapes=[pltpu.VMEM((8, D), jnp.float32)],
    )
    return pl.pallas_call(
        _body,
        grid_spec=grid_spec,
        out_shape=jax.ShapeDtypeStruct((BATCH, 1, D), jnp.float32),
        compiler_params=pltpu.CompilerParams(
            dimension_semantics=("arbitrary", "arbitrary"),
        ),
    )(lens, input2)


def kernel(input, sequence_lengths):
    lens = sequence_lengths.astype(jnp.int32)
    input2 = input.reshape(SEQ, BATCH * D)
    return _seq_mean_tc(input2, lens).reshape(BATCH, D)


# speedup vs baseline: 1.0245x; 1.0057x over previous
import jax
import jax.numpy as jnp
from jax import lax
from jax.experimental import pallas as pl
from jax.experimental.pallas import tpu as pltpu

SEQ = 2048
BATCH = 16
D = 1024
BT = 128
NT = SEQ // BT


def _body(lens_ref, in_ref, out_ref, acc_ref):
    b = pl.program_id(0)
    t = pl.program_id(1)
    L = lens_ref[b]
    nb = (L + (BT - 1)) // BT
    rem = L - t * BT

    @pl.when(t == 0)
    def _():
        acc_ref[...] = jnp.zeros_like(acc_ref)

    @pl.when(rem >= BT)
    def _():
        x = in_ref[...].reshape(BT // 8, 8, D)
        acc_ref[...] += jnp.sum(x, axis=0)

    @pl.when((t < nb) & (rem < BT))
    def _():
        x = in_ref[...].reshape(BT, D)
        row = lax.broadcasted_iota(jnp.int32, (BT, D), 0)
        masked = jnp.where(row < rem, x, 0.0).reshape(BT // 8, 8, D)
        acc_ref[...] += jnp.sum(masked, axis=0)

    @pl.when(t == nb - 1)
    def _():
        s = jnp.sum(acc_ref[...], axis=0)
        out_ref[...] = (s / L.astype(jnp.float32))[None, None, :]


def _seq_mean_tc(input2, lens):
    grid_spec = pltpu.PrefetchScalarGridSpec(
        num_scalar_prefetch=1,
        grid=(BATCH, NT),
        in_specs=[
            pl.BlockSpec(
                (BT, 1, 1, D),
                lambda b, t, lens: (
                    jnp.minimum(t, (lens[b] + (BT - 1)) // BT - 1), b, 0, 0),
            ),
        ],
        out_specs=pl.BlockSpec((1, 1, D), lambda b, t, lens: (b, 0, 0)),
        scratch_shapes=[pltpu.VMEM((8, D), jnp.float32)],
    )
    return pl.pallas_call(
        _body,
        grid_spec=grid_spec,
        out_shape=jax.ShapeDtypeStruct((BATCH, 1, D), jnp.float32),
        compiler_params=pltpu.CompilerParams(
            dimension_semantics=("arbitrary", "arbitrary"),
        ),
    )(lens, input2)


def kernel(input, sequence_lengths):
    lens = sequence_lengths.astype(jnp.int32)
    input4 = input.reshape(SEQ, BATCH, 1, D)
    return _seq_mean_tc(input4, lens).reshape(BATCH, D)
